# baseline (speedup 1.0000x reference)
import functools

import jax
import jax.numpy as jnp
from jax import lax
from jax.experimental import pallas as pl
from jax.experimental.pallas import tpu as pltpu
from jax.experimental.pallas import tpu_sc as plsc

BATCH = 16384
MF_DIM = 64
MLP_DIM = 128

_NC = 2
_NS = 16
_NW = _NC * _NS
_BPW = BATCH // _NW
_CHUNK = 256
_NCHUNK = _BPW // _CHUNK


def _sc_gather(user, item, mf_u, mf_i, mlp_u, mlp_i):
  mesh = plsc.VectorSubcoreMesh(
      core_axis_name="c", subcore_axis_name="s",
      num_cores=_NC, num_subcores=_NS)

  @functools.partial(
      pl.kernel,
      out_type=[
          jax.ShapeDtypeStruct((BATCH, MF_DIM), jnp.float32),
          jax.ShapeDtypeStruct((BATCH, MF_DIM), jnp.float32),
          jax.ShapeDtypeStruct((BATCH, MLP_DIM), jnp.float32),
          jax.ShapeDtypeStruct((BATCH, MLP_DIM), jnp.float32),
      ],
      mesh=mesh,
      compiler_params=pltpu.CompilerParams(use_tc_tiling_on_sc=False),
      scratch_types=[
          pltpu.VMEM((_CHUNK,), jnp.int32),
          pltpu.VMEM((_CHUNK,), jnp.int32),
          pltpu.VMEM((_CHUNK, MF_DIM), jnp.float32),
          pltpu.VMEM((_CHUNK, MF_DIM), jnp.float32),
          pltpu.VMEM((_CHUNK, MLP_DIM), jnp.float32),
          pltpu.VMEM((_CHUNK, MLP_DIM), jnp.float32),
          pltpu.SemaphoreType.DMA,
      ],
  )
  def k(user_h, item_h, mfu_h, mfi_h, mlpu_h, mlpi_h,
        omfu_h, omfi_h, omlpu_h, omlpi_h,
        uidx, iidx, bufa, bufb, bufc, bufd, sem):
    wid = lax.axis_index("s") * _NC + lax.axis_index("c")
    for c in range(_NCHUNK):
      base = wid * _BPW + c * _CHUNK
      pltpu.sync_copy(user_h.at[pl.ds(base, _CHUNK)], uidx)
      pltpu.sync_copy(item_h.at[pl.ds(base, _CHUNK)], iidx)
      ca = pltpu.async_copy(mfu_h.at[uidx], bufa, sem)
      cb = pltpu.async_copy(mfi_h.at[iidx], bufb, sem)
      cc = pltpu.async_copy(mlpu_h.at[uidx], bufc, sem)
      cd = pltpu.async_copy(mlpi_h.at[iidx], bufd, sem)
      ca.wait()
      pltpu.sync_copy(bufa, omfu_h.at[pl.ds(base, _CHUNK)])
      cb.wait()
      pltpu.sync_copy(bufb, omfi_h.at[pl.ds(base, _CHUNK)])
      cc.wait()
      pltpu.sync_copy(bufc, omlpu_h.at[pl.ds(base, _CHUNK)])
      cd.wait()
      pltpu.sync_copy(bufd, omlpi_h.at[pl.ds(base, _CHUNK)])

  return k(user, item, mf_u, mf_i, mlp_u, mlp_i)


_BT = 2048


def _tc_body(mfu, mfi, mlpu, mlpi, w1u, w1i, b1, w2, b2, wo, bo, out):
  x = jnp.dot(mlpu[...], w1u[...], preferred_element_type=jnp.float32)
  x = x + jnp.dot(mlpi[...], w1i[...], preferred_element_type=jnp.float32)
  h1 = jnp.maximum(x + b1[...], 0.0)
  h2 = jnp.maximum(
      jnp.dot(h1, w2[...], preferred_element_type=jnp.float32) + b2[...], 0.0)
  g = mfu[...] * mfi[...]
  p = jnp.concatenate([g, h2], axis=1)
  z = jnp.sum(p * wo[...], axis=1, keepdims=True) + bo[...]
  out[...] = jax.nn.sigmoid(z)


def _tc_mlp(mfu, mfi, mlpu, mlpi, W1, b1, W2, b2, W_out, b_out):
  w1t = W1.T
  w1u = w1t[:MLP_DIM]
  w1i = w1t[MLP_DIM:]
  w2t = W2.T
  b1r = b1.reshape(1, -1)
  b2r = b2.reshape(1, -1)
  wo = W_out.reshape(1, -1)
  bo = b_out.reshape(1, 1)

  grid = (BATCH // _BT,)
  bspec_row = lambda d: pl.BlockSpec((_BT, d), lambda i: (i, 0))
  bspec_full = lambda s: pl.BlockSpec(s, lambda i: (0, 0))
  return pl.pallas_call(
      _tc_body,
      grid=grid,
      in_specs=[
          bspec_row(MF_DIM), bspec_row(MF_DIM),
          bspec_row(MLP_DIM), bspec_row(MLP_DIM),
          bspec_full((MLP_DIM, 128)), bspec_full((MLP_DIM, 128)),
          bspec_full((1, 128)),
          bspec_full((128, 64)), bspec_full((1, 64)),
          bspec_full((1, 128)), bspec_full((1, 1)),
      ],
      out_specs=pl.BlockSpec((_BT, 1), lambda i: (i, 0)),
      out_shape=jax.ShapeDtypeStruct((BATCH, 1), jnp.float32),
      compiler_params=pltpu.CompilerParams(
          dimension_semantics=("arbitrary",)),
  )(mfu, mfi, mlpu, mlpi, w1u, w1i, b1r, w2t, b2r, wo, bo)


def kernel(user, item, mf_emb_user, mf_emb_item, mlp_emb_user, mlp_emb_item,
           W1, b1, W2, b2, W_out, b_out):
  user = user.astype(jnp.int32)
  item = item.astype(jnp.int32)
  mfu, mfi, mlpu, mlpi = _sc_gather(
      user, item, mf_emb_user, mf_emb_item, mlp_emb_user, mlp_emb_item)
  return _tc_mlp(mfu, mfi, mlpu, mlpi, W1, b1, W2, b2, W_out, b_out)


# speedup vs baseline: 1.1375x; 1.1375x over previous
import functools

import jax
import jax.numpy as jnp
from jax import lax
from jax.experimental import pallas as pl
from jax.experimental.pallas import tpu as pltpu
from jax.experimental.pallas import tpu_sc as plsc

BATCH = 16384
MF_DIM = 64
MLP_DIM = 128

_NC = 2
_NS = 16
_NW = _NC * _NS
_BPW = BATCH // _NW
_CHUNK = 128
_NCHUNK = _BPW // _CHUNK


def _sc_gather(user, item, mf_u, mf_i, mlp_u, mlp_i):
  mesh = plsc.VectorSubcoreMesh(
      core_axis_name="c", subcore_axis_name="s",
      num_cores=_NC, num_subcores=_NS)

  @functools.partial(
      pl.kernel,
      out_type=[
          jax.ShapeDtypeStruct((BATCH, 2 * MF_DIM), jnp.float32),
          jax.ShapeDtypeStruct((BATCH, MLP_DIM), jnp.float32),
          jax.ShapeDtypeStruct((BATCH, MLP_DIM), jnp.float32),
      ],
      mesh=mesh,
      compiler_params=pltpu.CompilerParams(use_tc_tiling_on_sc=False),
      scratch_types=[
          pltpu.VMEM((_BPW,), jnp.int32),
          pltpu.VMEM((_BPW,), jnp.int32),
          pltpu.VMEM((2, _CHUNK, MF_DIM), jnp.float32),
          pltpu.VMEM((2, _CHUNK, MF_DIM), jnp.float32),
          pltpu.VMEM((2, _CHUNK, MLP_DIM), jnp.float32),
          pltpu.VMEM((2, _CHUNK, MLP_DIM), jnp.float32),
          pltpu.SemaphoreType.DMA,
          pltpu.SemaphoreType.DMA,
      ],
  )
  def k(user_h, item_h, mfu_h, mfi_h, mlpu_h, mlpi_h,
        omf_h, omlpu_h, omlpi_h,
        uidx, iidx, bufa, bufb, bufc, bufd, gsem, wsem):
    wid = lax.axis_index("s") * _NC + lax.axis_index("c")
    pltpu.sync_copy(user_h.at[pl.ds(wid * _BPW, _BPW)], uidx)
    pltpu.sync_copy(item_h.at[pl.ds(wid * _BPW, _BPW)], iidx)
    writes = [None, None]
    for c in range(_NCHUNK):
      b = c % 2
      base = wid * _BPW + c * _CHUNK
      if writes[b] is not None:
        for w in writes[b]:
          w.wait()
      uc = uidx.at[pl.ds(c * _CHUNK, _CHUNK)]
      ic = iidx.at[pl.ds(c * _CHUNK, _CHUNK)]
      ca = pltpu.async_copy(mfu_h.at[uc], bufa.at[b], gsem)
      cb = pltpu.async_copy(mfi_h.at[ic], bufb.at[b], gsem)
      cc = pltpu.async_copy(mlpu_h.at[uc], bufc.at[b], gsem)
      cd = pltpu.async_copy(mlpi_h.at[ic], bufd.at[b], gsem)
      ca.wait()
      wa = pltpu.async_copy(
          bufa.at[b], omf_h.at[pl.ds(base, _CHUNK), pl.ds(0, MF_DIM)], wsem)
      cb.wait()
      wb = pltpu.async_copy(
          bufb.at[b], omf_h.at[pl.ds(base, _CHUNK), pl.ds(MF_DIM, MF_DIM)],
          wsem)
      cc.wait()
      wc = pltpu.async_copy(bufc.at[b], omlpu_h.at[pl.ds(base, _CHUNK)], wsem)
      cd.wait()
      wd = pltpu.async_copy(bufd.at[b], omlpi_h.at[pl.ds(base, _CHUNK)], wsem)
      writes[b] = (wa, wb, wc, wd)
    for ws in writes:
      for w in ws:
        w.wait()

  return k(user, item, mf_u, mf_i, mlp_u, mlp_i)


_BT = 2048


def _tc_body(mf, mlpu, mlpi, w1u, w1i, b1, w2, b2, wo, bo, out):
  x = jnp.dot(mlpu[...], w1u[...], preferred_element_type=jnp.float32)
  x = x + jnp.dot(mlpi[...], w1i[...], preferred_element_type=jnp.float32)
  h1 = jnp.maximum(x + b1[...], 0.0)
  h2 = jnp.maximum(
      jnp.dot(h1, w2[...], preferred_element_type=jnp.float32) + b2[...], 0.0)
  mfb = mf[...]
  g = mfb[:, :MF_DIM] * mfb[:, MF_DIM:]
  p = jnp.concatenate([g, h2], axis=1)
  z = jnp.sum(p * wo[...], axis=1, keepdims=True) + bo[...]
  out[...] = jax.nn.sigmoid(z)


def _tc_mlp(mf, mlpu, mlpi, W1, b1, W2, b2, W_out, b_out):
  w1t = W1.T
  w1u = w1t[:MLP_DIM]
  w1i = w1t[MLP_DIM:]
  w2t = W2.T
  b1r = b1.reshape(1, -1)
  b2r = b2.reshape(1, -1)
  wo = W_out.reshape(1, -1)
  bo = b_out.reshape(1, 1)

  grid = (BATCH // _BT,)
  bspec_row = lambda d: pl.BlockSpec((_BT, d), lambda i: (i, 0))
  bspec_full = lambda s: pl.BlockSpec(s, lambda i: (0, 0))
  return pl.pallas_call(
      _tc_body,
      grid=grid,
      in_specs=[
          bspec_row(2 * MF_DIM),
          bspec_row(MLP_DIM), bspec_row(MLP_DIM),
          bspec_full((MLP_DIM, 128)), bspec_full((MLP_DIM, 128)),
          bspec_full((1, 128)),
          bspec_full((128, 64)), bspec_full((1, 64)),
          bspec_full((1, 128)), bspec_full((1, 1)),
      ],
      out_specs=pl.BlockSpec((_BT, 1), lambda i: (i, 0)),
      out_shape=jax.ShapeDtypeStruct((BATCH, 1), jnp.float32),
      compiler_params=pltpu.CompilerParams(
          dimension_semantics=("arbitrary",)),
  )(mf, mlpu, mlpi, w1u, w1i, b1r, w2t, b2r, wo, bo)


def kernel(user, item, mf_emb_user, mf_emb_item, mlp_emb_user, mlp_emb_item,
           W1, b1, W2, b2, W_out, b_out):
  user = user.astype(jnp.int32)
  item = item.astype(jnp.int32)
  mf, mlpu, mlpi = _sc_gather(
      user, item, mf_emb_user, mf_emb_item, mlp_emb_user, mlp_emb_item)
  return _tc_mlp(mf, mlpu, mlpi, W1, b1, W2, b2, W_out, b_out)


# speedup vs baseline: 1.1560x; 1.0162x over previous
import functools

import jax
import jax.numpy as jnp
from jax import lax
from jax.experimental import pallas as pl
from jax.experimental.pallas import tpu as pltpu
from jax.experimental.pallas import tpu_sc as plsc

BATCH = 16384
MF_DIM = 64
MLP_DIM = 128

_NC = 2
_NS = 16
_NW = _NC * _NS
_BPW = BATCH // _NW
_CHUNK = 128
_NCHUNK = _BPW // _CHUNK


_MESH = plsc.VectorSubcoreMesh(
    core_axis_name="c", subcore_axis_name="s",
    num_cores=_NC, num_subcores=_NS)


def _sc_gather_mlp(user, item, mlp_u, mlp_i):
  @functools.partial(
      pl.kernel,
      out_type=[
          jax.ShapeDtypeStruct((BATCH, MLP_DIM), jnp.float32),
          jax.ShapeDtypeStruct((BATCH, MLP_DIM), jnp.float32),
      ],
      mesh=_MESH,
      scratch_types=[
          pltpu.VMEM((_BPW,), jnp.int32),
          pltpu.VMEM((_BPW,), jnp.int32),
          pltpu.VMEM((2, _CHUNK, MLP_DIM), jnp.float32),
          pltpu.VMEM((2, _CHUNK, MLP_DIM), jnp.float32),
          pltpu.SemaphoreType.DMA,
          pltpu.SemaphoreType.DMA,
      ],
  )
  def k(user_h, item_h, mlpu_h, mlpi_h, omlpu_h, omlpi_h,
        uidx, iidx, bufc, bufd, gsem, wsem):
    wid = lax.axis_index("s") * _NC + lax.axis_index("c")
    pltpu.sync_copy(user_h.at[pl.ds(wid * _BPW, _BPW)], uidx)
    pltpu.sync_copy(item_h.at[pl.ds(wid * _BPW, _BPW)], iidx)
    writes = [None, None]
    for c in range(_NCHUNK):
      b = c % 2
      base = wid * _BPW + c * _CHUNK
      if writes[b] is not None:
        for w in writes[b]:
          w.wait()
      uc = uidx.at[pl.ds(c * _CHUNK, _CHUNK)]
      ic = iidx.at[pl.ds(c * _CHUNK, _CHUNK)]
      cc = pltpu.async_copy(mlpu_h.at[uc], bufc.at[b], gsem)
      cd = pltpu.async_copy(mlpi_h.at[ic], bufd.at[b], gsem)
      cc.wait()
      wc = pltpu.async_copy(bufc.at[b], omlpu_h.at[pl.ds(base, _CHUNK)], wsem)
      cd.wait()
      wd = pltpu.async_copy(bufd.at[b], omlpi_h.at[pl.ds(base, _CHUNK)], wsem)
      writes[b] = (wc, wd)
    for ws in writes:
      for w in ws:
        w.wait()

  return k(user, item, mlp_u, mlp_i)


def _sc_gather_mf(user, item, mf_u, mf_i):
  @functools.partial(
      pl.kernel,
      out_type=jax.ShapeDtypeStruct((BATCH, 2 * MF_DIM), jnp.float32),
      mesh=_MESH,
      compiler_params=pltpu.CompilerParams(use_tc_tiling_on_sc=False),
      scratch_types=[
          pltpu.VMEM((_BPW,), jnp.int32),
          pltpu.VMEM((_BPW,), jnp.int32),
          pltpu.VMEM((2, _CHUNK, MF_DIM), jnp.float32),
          pltpu.VMEM((2, _CHUNK, MF_DIM), jnp.float32),
          pltpu.SemaphoreType.DMA,
          pltpu.SemaphoreType.DMA,
      ],
  )
  def k(user_h, item_h, mfu_h, mfi_h, omf_h,
        uidx, iidx, bufa, bufb, gsem, wsem):
    wid = lax.axis_index("s") * _NC + lax.axis_index("c")
    pltpu.sync_copy(user_h.at[pl.ds(wid * _BPW, _BPW)], uidx)
    pltpu.sync_copy(item_h.at[pl.ds(wid * _BPW, _BPW)], iidx)
    writes = [None, None]
    for c in range(_NCHUNK):
      b = c % 2
      base = wid * _BPW + c * _CHUNK
      if writes[b] is not None:
        for w in writes[b]:
          w.wait()
      uc = uidx.at[pl.ds(c * _CHUNK, _CHUNK)]
      ic = iidx.at[pl.ds(c * _CHUNK, _CHUNK)]
      ca = pltpu.async_copy(mfu_h.at[uc], bufa.at[b], gsem)
      cb = pltpu.async_copy(mfi_h.at[ic], bufb.at[b], gsem)
      ca.wait()
      wa = pltpu.async_copy(
          bufa.at[b], omf_h.at[pl.ds(base, _CHUNK), pl.ds(0, MF_DIM)], wsem)
      cb.wait()
      wb = pltpu.async_copy(
          bufb.at[b], omf_h.at[pl.ds(base, _CHUNK), pl.ds(MF_DIM, MF_DIM)],
          wsem)
      writes[b] = (wa, wb)
    for ws in writes:
      for w in ws:
        w.wait()

  return k(user, item, mf_u, mf_i)


_BT = 2048


def _tc_body(mf, mlpu, mlpi, w1u, w1i, b1, w2, b2, wo, bo, out):
  x = jnp.dot(mlpu[...], w1u[...], preferred_element_type=jnp.float32)
  x = x + jnp.dot(mlpi[...], w1i[...], preferred_element_type=jnp.float32)
  h1 = jnp.maximum(x + b1[...], 0.0)
  h2 = jnp.maximum(
      jnp.dot(h1, w2[...], preferred_element_type=jnp.float32) + b2[...], 0.0)
  mfb = mf[...]
  g = mfb[:, :MF_DIM] * mfb[:, MF_DIM:]
  p = jnp.concatenate([g, h2], axis=1)
  z = jnp.sum(p * wo[...], axis=1, keepdims=True) + bo[...]
  out[...] = jax.nn.sigmoid(z)


def _tc_mlp(mf, mlpu, mlpi, W1, b1, W2, b2, W_out, b_out):
  w1t = W1.T
  w1u = w1t[:MLP_DIM]
  w1i = w1t[MLP_DIM:]
  w2t = W2.T
  b1r = b1.reshape(1, -1)
  b2r = b2.reshape(1, -1)
  wo = W_out.reshape(1, -1)
  bo = b_out.reshape(1, 1)

  grid = (BATCH // _BT,)
  bspec_row = lambda d: pl.BlockSpec((_BT, d), lambda i: (i, 0))
  bspec_full = lambda s: pl.BlockSpec(s, lambda i: (0, 0))
  return pl.pallas_call(
      _tc_body,
      grid=grid,
      in_specs=[
          bspec_row(2 * MF_DIM),
          bspec_row(MLP_DIM), bspec_row(MLP_DIM),
          bspec_full((MLP_DIM, 128)), bspec_full((MLP_DIM, 128)),
          bspec_full((1, 128)),
          bspec_full((128, 64)), bspec_full((1, 64)),
          bspec_full((1, 128)), bspec_full((1, 1)),
      ],
      out_specs=pl.BlockSpec((_BT, 1), lambda i: (i, 0)),
      out_shape=jax.ShapeDtypeStruct((BATCH, 1), jnp.float32),
      compiler_params=pltpu.CompilerParams(
          dimension_semantics=("arbitrary",)),
  )(mf, mlpu, mlpi, w1u, w1i, b1r, w2t, b2r, wo, bo)


def kernel(user, item, mf_emb_user, mf_emb_item, mlp_emb_user, mlp_emb_item,
           W1, b1, W2, b2, W_out, b_out):
  user = user.astype(jnp.int32)
  item = item.astype(jnp.int32)
  mlpu, mlpi = _sc_gather_mlp(user, item, mlp_emb_user, mlp_emb_item)
  mf = _sc_gather_mf(user, item, mf_emb_user, mf_emb_item)
  return _tc_mlp(mf, mlpu, mlpi, W1, b1, W2, b2, W_out, b_out)


# speedup vs baseline: 1.2242x; 1.0591x over previous
import functools

import jax
import jax.numpy as jnp
from jax import lax
from jax.experimental import pallas as pl
from jax.experimental.pallas import tpu as pltpu
from jax.experimental.pallas import tpu_sc as plsc

BATCH = 16384
MF_DIM = 64
MLP_DIM = 128

_NC = 2
_NS = 16
_NW = _NC * _NS
_BPW = BATCH // _NW
_CHUNK = 128
_NCHUNK = _BPW // _CHUNK


_MESH = plsc.VectorSubcoreMesh(
    core_axis_name="c", subcore_axis_name="s",
    num_cores=_NC, num_subcores=_NS)


def _sc_gather_mlp(user, item, mlp_u, mlp_i):
  @functools.partial(
      pl.kernel,
      out_type=[
          jax.ShapeDtypeStruct((BATCH, MLP_DIM), jnp.float32),
          jax.ShapeDtypeStruct((BATCH, MLP_DIM), jnp.float32),
      ],
      mesh=_MESH,
      scratch_types=[
          pltpu.VMEM((_BPW,), jnp.int32),
          pltpu.VMEM((_BPW,), jnp.int32),
          pltpu.VMEM((2, _CHUNK, MLP_DIM), jnp.float32),
          pltpu.VMEM((2, _CHUNK, MLP_DIM), jnp.float32),
          pltpu.SemaphoreType.DMA,
          pltpu.SemaphoreType.DMA,
      ],
  )
  def k(user_h, item_h, mlpu_h, mlpi_h, omlpu_h, omlpi_h,
        uidx, iidx, bufc, bufd, gsem, wsem):
    wid = lax.axis_index("s") * _NC + lax.axis_index("c")
    pltpu.sync_copy(user_h.at[pl.ds(wid * _BPW, _BPW)], uidx)
    pltpu.sync_copy(item_h.at[pl.ds(wid * _BPW, _BPW)], iidx)
    writes = [None, None]
    for c in range(_NCHUNK):
      b = c % 2
      base = wid * _BPW + c * _CHUNK
      if writes[b] is not None:
        for w in writes[b]:
          w.wait()
      uc = uidx.at[pl.ds(c * _CHUNK, _CHUNK)]
      ic = iidx.at[pl.ds(c * _CHUNK, _CHUNK)]
      cc = pltpu.async_copy(mlpu_h.at[uc], bufc.at[b], gsem)
      cd = pltpu.async_copy(mlpi_h.at[ic], bufd.at[b], gsem)
      cc.wait()
      wc = pltpu.async_copy(bufc.at[b], omlpu_h.at[pl.ds(base, _CHUNK)], wsem)
      cd.wait()
      wd = pltpu.async_copy(bufd.at[b], omlpi_h.at[pl.ds(base, _CHUNK)], wsem)
      writes[b] = (wc, wd)
    for ws in writes:
      for w in ws:
        w.wait()

  return k(user, item, mlp_u, mlp_i)


def _sc_gather_mf(user, item, mfcat):
  @functools.partial(
      pl.kernel,
      out_type=[
          jax.ShapeDtypeStruct((BATCH, 2 * MF_DIM), jnp.float32),
          jax.ShapeDtypeStruct((BATCH, 2 * MF_DIM), jnp.float32),
      ],
      mesh=_MESH,
      scratch_types=[
          pltpu.VMEM((_BPW,), jnp.int32),
          pltpu.VMEM((_BPW,), jnp.int32),
          pltpu.VMEM((2, _CHUNK, 2 * MF_DIM), jnp.float32),
          pltpu.VMEM((2, _CHUNK, 2 * MF_DIM), jnp.float32),
          pltpu.SemaphoreType.DMA,
          pltpu.SemaphoreType.DMA,
      ],
  )
  def k(user_h, item_h, mfcat_h, omfu_h, omfi_h,
        uidx, iidx, bufa, bufb, gsem, wsem):
    wid = lax.axis_index("s") * _NC + lax.axis_index("c")
    pltpu.sync_copy(user_h.at[pl.ds(wid * _BPW, _BPW)], uidx)
    pltpu.sync_copy(item_h.at[pl.ds(wid * _BPW, _BPW)], iidx)
    writes = [None, None]
    for c in range(_NCHUNK):
      b = c % 2
      base = wid * _BPW + c * _CHUNK
      if writes[b] is not None:
        for w in writes[b]:
          w.wait()
      uc = uidx.at[pl.ds(c * _CHUNK, _CHUNK)]
      ic = iidx.at[pl.ds(c * _CHUNK, _CHUNK)]
      ca = pltpu.async_copy(mfcat_h.at[uc], bufa.at[b], gsem)
      cb = pltpu.async_copy(mfcat_h.at[ic], bufb.at[b], gsem)
      ca.wait()
      wa = pltpu.async_copy(bufa.at[b], omfu_h.at[pl.ds(base, _CHUNK)], wsem)
      cb.wait()
      wb = pltpu.async_copy(bufb.at[b], omfi_h.at[pl.ds(base, _CHUNK)], wsem)
      writes[b] = (wa, wb)
    for ws in writes:
      for w in ws:
        w.wait()

  return k(user, item, mfcat)


_BT = 2048


def _tc_body(mfu, mfi, mlpu, mlpi, w1u, w1i, b1, w2, b2, wo, bo, out):
  x = jnp.dot(mlpu[...], w1u[...], preferred_element_type=jnp.float32)
  x = x + jnp.dot(mlpi[...], w1i[...], preferred_element_type=jnp.float32)
  h1 = jnp.maximum(x + b1[...], 0.0)
  h2 = jnp.maximum(
      jnp.dot(h1, w2[...], preferred_element_type=jnp.float32) + b2[...], 0.0)
  g = mfu[...][:, :MF_DIM] * mfi[...][:, MF_DIM:]
  p = jnp.concatenate([g, h2], axis=1)
  z = jnp.sum(p * wo[...], axis=1, keepdims=True) + bo[...]
  out[...] = jax.nn.sigmoid(z)


def _tc_mlp(mfu, mfi, mlpu, mlpi, W1, b1, W2, b2, W_out, b_out):
  w1t = W1.T
  w1u = w1t[:MLP_DIM]
  w1i = w1t[MLP_DIM:]
  w2t = W2.T
  b1r = b1.reshape(1, -1)
  b2r = b2.reshape(1, -1)
  wo = W_out.reshape(1, -1)
  bo = b_out.reshape(1, 1)

  grid = (BATCH // _BT,)
  bspec_row = lambda d: pl.BlockSpec((_BT, d), lambda i: (i, 0))
  bspec_full = lambda s: pl.BlockSpec(s, lambda i: (0, 0))
  return pl.pallas_call(
      _tc_body,
      grid=grid,
      in_specs=[
          bspec_row(2 * MF_DIM), bspec_row(2 * MF_DIM),
          bspec_row(MLP_DIM), bspec_row(MLP_DIM),
          bspec_full((MLP_DIM, 128)), bspec_full((MLP_DIM, 128)),
          bspec_full((1, 128)),
          bspec_full((128, 64)), bspec_full((1, 64)),
          bspec_full((1, 128)), bspec_full((1, 1)),
      ],
      out_specs=pl.BlockSpec((_BT, 1), lambda i: (i, 0)),
      out_shape=jax.ShapeDtypeStruct((BATCH, 1), jnp.float32),
      compiler_params=pltpu.CompilerParams(
          dimension_semantics=("arbitrary",)),
  )(mfu, mfi, mlpu, mlpi, w1u, w1i, b1r, w2t, b2r, wo, bo)


def kernel(user, item, mf_emb_user, mf_emb_item, mlp_emb_user, mlp_emb_item,
           W1, b1, W2, b2, W_out, b_out):
  user = user.astype(jnp.int32)
  item = item.astype(jnp.int32)
  mfcat = jnp.concatenate([mf_emb_user, mf_emb_item], axis=1)
  mlpu, mlpi = _sc_gather_mlp(user, item, mlp_emb_user, mlp_emb_item)
  mfu, mfi = _sc_gather_mf(user, item, mfcat)
  return _tc_mlp(mfu, mfi, mlpu, mlpi, W1, b1, W2, b2, W_out, b_out)
